# baseline (speedup 1.0000x reference)
import functools

import jax
import jax.numpy as jnp
from jax import lax
from jax.experimental import pallas as pl
from jax.experimental.pallas import tpu as pltpu
from jax.experimental.pallas import tpu_sc as plsc

N = 10000
E = 320000
D = 128
NC = 2
NS = 16
K = 500
EPS = E // (NC * NS)
CH = EPS // K
NP = 10240
RT = NP // NS
ZR = 80


BLK = 1000


def _matmul_body(h0, h1, h2, w0, w1, w2, o_ref):
    for g, (h, w) in enumerate(((h0, w0), (h1, w1), (h2, w2))):
        o_ref[g] = jnp.dot(h[...], w[...],
                           preferred_element_type=jnp.float32
                           ).astype(jnp.bfloat16)


def _matmul(h0, h1, h2, w0, w1, w2):
    return pl.pallas_call(
        _matmul_body,
        grid=(N // BLK,),
        in_specs=[pl.BlockSpec((BLK, D), lambda i: (i, 0))] * 3
        + [pl.BlockSpec((D, D), lambda i: (0, 0))] * 3,
        out_specs=pl.BlockSpec((3, BLK, D), lambda i: (0, i, 0)),
        out_shape=jax.ShapeDtypeStruct((3, N, D), jnp.bfloat16),
    )(h0, h1, h2, w0, w1, w2)


def _finalize_body(p_ref, h0, h1, h2, b_ref, o0, o1, o2):
    for g, (h, o) in enumerate(((h0, o0), (h1, o1), (h2, o2))):
        agg = (p_ref[g, 0].astype(jnp.float32)
               + p_ref[g, 1].astype(jnp.float32) + b_ref[g, 0][None, :])
        o[...] = jnp.maximum(agg, 0.0) + h[...]


def _finalize(parts, h0, h1, h2, bs):
    return pl.pallas_call(
        _finalize_body,
        grid=(N // BLK,),
        in_specs=[pl.BlockSpec((3, 2, BLK, D), lambda i: (0, 0, i, 0))]
        + [pl.BlockSpec((BLK, D), lambda i: (i, 0))] * 3
        + [pl.BlockSpec((3, 1, D), lambda i: (0, 0, 0))],
        out_specs=[pl.BlockSpec((BLK, D), lambda i: (i, 0))] * 3,
        out_shape=[jax.ShapeDtypeStruct((N, D), jnp.float32)] * 3,
    )(parts, h0, h1, h2, bs)


def _sc_body(hw0, hw1, hw2, e0, e1, e2, po,
             acc, sidx, didx, rows_a, rows_b, sem_a, sem_b):
    cid = lax.axis_index("c")
    sid = lax.axis_index("s")
    wid = cid * NS + sid

    for g, (hw, e) in enumerate(((hw0, e0), (hw1, e1), (hw2, e2))):
        def zlp(i, c):
            rows_a[i // (D // 32), pl.ds((i % (D // 32)) * 32, 32)] = (
                jnp.zeros((32,), jnp.bfloat16))
            return c
        lax.fori_loop(0, ZR * (D // 32), zlp, 0)

        def zero(z, c):
            pltpu.sync_copy(rows_a.at[pl.ds(0, ZR)],
                            acc.at[pl.ds(sid * RT + z * ZR, ZR)])
            return c
        lax.fori_loop(0, RT // ZR, zero, 0)
        plsc.subcore_barrier()

        def _start(ch, buf, sem):
            pltpu.async_copy(hw.at[sidx.at[ch]], buf, sem)

        def _wait(ch, buf, sem):
            pltpu.make_async_copy(hw.at[sidx.at[ch]], buf, sem).wait()

        def _scat(ch, buf):
            pltpu.sync_copy(buf, acc.at[didx.at[ch]], add=True)

        pltpu.sync_copy(e.at[0, wid], sidx)
        pltpu.sync_copy(e.at[1, wid], didx)
        _start(0, rows_a, sem_a)

        def chunk(i, c):
            _start(2 * i + 1, rows_b, sem_b)
            _wait(2 * i, rows_a, sem_a)
            _scat(2 * i, rows_a)
            _start(2 * i + 2, rows_a, sem_a)
            _wait(2 * i + 1, rows_b, sem_b)
            _scat(2 * i + 1, rows_b)
            return c
        lax.fori_loop(0, CH // 2 - 1, chunk, 0)
        _start(CH - 1, rows_b, sem_b)
        _wait(CH - 2, rows_a, sem_a)
        _scat(CH - 2, rows_a)
        _wait(CH - 1, rows_b, sem_b)
        _scat(CH - 1, rows_b)
        plsc.subcore_barrier()

        def wout(z, c):
            r0 = sid * RT + 2 * z * ZR
            r1 = r0 + ZR
            pltpu.sync_copy(acc.at[pl.ds(r0, ZR)], rows_a.at[pl.ds(0, ZR)])
            pltpu.async_copy(rows_a.at[pl.ds(0, ZR)],
                             po.at[g, cid, pl.ds(r0, ZR)], sem_a)
            pltpu.sync_copy(acc.at[pl.ds(r1, ZR)], rows_b.at[pl.ds(0, ZR)])
            pltpu.async_copy(rows_b.at[pl.ds(0, ZR)],
                             po.at[g, cid, pl.ds(r1, ZR)], sem_b)
            pltpu.make_async_copy(rows_a.at[pl.ds(0, ZR)],
                                  po.at[g, cid, pl.ds(r0, ZR)],
                                  sem_a).wait()
            pltpu.make_async_copy(rows_b.at[pl.ds(0, ZR)],
                                  po.at[g, cid, pl.ds(r1, ZR)],
                                  sem_b).wait()
            return c
        lax.fori_loop(0, RT // ZR // 2, wout, 0)


_sc_call = pl.kernel(
    _sc_body,
    out_type=jax.ShapeDtypeStruct((3, NC, NP, D), jnp.bfloat16),
    mesh=plsc.VectorSubcoreMesh(core_axis_name="c", subcore_axis_name="s"),
    compiler_params=pltpu.CompilerParams(use_tc_tiling_on_sc=False),
    scratch_types=[
        pltpu.VMEM_SHARED((NP, D), jnp.bfloat16),
        pltpu.VMEM((CH, K), jnp.int32),
        pltpu.VMEM((CH, K), jnp.int32),
        pltpu.VMEM((K, D), jnp.bfloat16),
        pltpu.VMEM((K, D), jnp.bfloat16),
        pltpu.SemaphoreType.DMA,
        pltpu.SemaphoreType.DMA,
    ],
)


def kernel(h_mf_new, h_bp_new, h_cc_new, mf_edge_index, bp_edge_index,
           cc_edge_index, W_mf, b_mf, W_bp, b_bp, W_cc, b_cc):
    bs = jnp.stack([b_mf, b_bp, b_cc]).reshape(3, 1, D)

    hw = _matmul(h_mf_new, h_bp_new, h_cc_new, W_mf, W_bp, W_cc)

    def _idx(ei):
        return ei.astype(jnp.int32).reshape(2, NC * NS, CH, K)

    parts = _sc_call(hw[0], hw[1], hw[2], _idx(mf_edge_index),
                     _idx(bp_edge_index), _idx(cc_edge_index))

    return tuple(_finalize(parts, h_mf_new, h_bp_new, h_cc_new, bs))


# speedup vs baseline: 1.0530x; 1.0530x over previous
import functools

import jax
import jax.numpy as jnp
from jax import lax
from jax.experimental import pallas as pl
from jax.experimental.pallas import tpu as pltpu
from jax.experimental.pallas import tpu_sc as plsc

N = 10000
E = 320000
D = 128
NC = 2
NS = 16
K = 250
EPS = E // (NC * NS)
CH = EPS // K
CH2 = CH // 2
NP = 10240
RT = NP // NS
ZR = 80


BLK = 1000


def _matmul_body(h0, h1, h2, w0, w1, w2, o_ref):
    for g, (h, w) in enumerate(((h0, w0), (h1, w1), (h2, w2))):
        o_ref[g] = jnp.dot(h[...], w[...],
                           preferred_element_type=jnp.float32
                           ).astype(jnp.bfloat16)


def _matmul(h0, h1, h2, w0, w1, w2):
    return pl.pallas_call(
        _matmul_body,
        grid=(N // BLK,),
        in_specs=[pl.BlockSpec((BLK, D), lambda i: (i, 0))] * 3
        + [pl.BlockSpec((D, D), lambda i: (0, 0))] * 3,
        out_specs=pl.BlockSpec((3, BLK, D), lambda i: (0, i, 0)),
        out_shape=jax.ShapeDtypeStruct((3, N, D), jnp.bfloat16),
    )(h0, h1, h2, w0, w1, w2)


def _finalize_body(p_ref, h0, h1, h2, b_ref, o0, o1, o2):
    for g, (h, o) in enumerate(((h0, o0), (h1, o1), (h2, o2))):
        agg = (p_ref[g, 0].astype(jnp.float32)
               + p_ref[g, 1].astype(jnp.float32) + b_ref[g, 0][None, :])
        o[...] = jnp.maximum(agg, 0.0) + h[...]


def _finalize(parts, h0, h1, h2, bs):
    return pl.pallas_call(
        _finalize_body,
        grid=(N // BLK,),
        in_specs=[pl.BlockSpec((3, 2, BLK, D), lambda i: (0, 0, i, 0))]
        + [pl.BlockSpec((BLK, D), lambda i: (i, 0))] * 3
        + [pl.BlockSpec((3, 1, D), lambda i: (0, 0, 0))],
        out_specs=[pl.BlockSpec((BLK, D), lambda i: (i, 0))] * 3,
        out_shape=[jax.ShapeDtypeStruct((N, D), jnp.float32)] * 3,
    )(parts, h0, h1, h2, bs)


def _sc_body(hw0, hw1, hw2, e0, e1, e2, po,
             acc, sidx, didx, rows_a, rows_b, rows_c, sem_a, sem_b, sem_c):
    cid = lax.axis_index("c")
    sid = lax.axis_index("s")
    wid = cid * NS + sid

    for g, (hw, e) in enumerate(((hw0, e0), (hw1, e1), (hw2, e2))):
        def zlp(i, c):
            rows_a[i // (D // 32), pl.ds((i % (D // 32)) * 32, 32)] = (
                jnp.zeros((32,), jnp.bfloat16))
            return c
        lax.fori_loop(0, ZR * (D // 32), zlp, 0)

        def zero(z, c):
            pltpu.sync_copy(rows_a.at[pl.ds(0, ZR)],
                            acc.at[pl.ds(sid * RT + z * ZR, ZR)])
            return c
        lax.fori_loop(0, RT // ZR, zero, 0)
        plsc.subcore_barrier()

        def _start(ch, buf, sem):
            pltpu.async_copy(hw.at[sidx.at[ch]], buf, sem)

        def _wait(ch, buf, sem):
            pltpu.make_async_copy(hw.at[sidx.at[ch]], buf, sem).wait()

        def _scat(ch, buf):
            pltpu.sync_copy(buf, acc.at[didx.at[ch]], add=True)

        for h in range(2):
            pltpu.sync_copy(e.at[0, wid, pl.ds(h * CH2, CH2)], sidx)
            pltpu.sync_copy(e.at[1, wid, pl.ds(h * CH2, CH2)], didx)
            _start(0, rows_a, sem_a)
            _start(1, rows_b, sem_b)

            def chunk(i, c):
                _start(3 * i + 2, rows_c, sem_c)
                _wait(3 * i, rows_a, sem_a)
                _scat(3 * i, rows_a)
                _start(3 * i + 3, rows_a, sem_a)
                _wait(3 * i + 1, rows_b, sem_b)
                _scat(3 * i + 1, rows_b)
                _start(3 * i + 4, rows_b, sem_b)
                _wait(3 * i + 2, rows_c, sem_c)
                _scat(3 * i + 2, rows_c)
                return c
            lax.fori_loop(0, (CH2 - 2) // 3, chunk, 0)
            _wait(CH2 - 2, rows_a, sem_a)
            _scat(CH2 - 2, rows_a)
            _wait(CH2 - 1, rows_b, sem_b)
            _scat(CH2 - 1, rows_b)
        plsc.subcore_barrier()

        def wout(z, c):
            r0 = sid * RT + 2 * z * ZR
            r1 = r0 + ZR
            pltpu.sync_copy(acc.at[pl.ds(r0, ZR)], rows_a.at[pl.ds(0, ZR)])
            pltpu.async_copy(rows_a.at[pl.ds(0, ZR)],
                             po.at[g, cid, pl.ds(r0, ZR)], sem_a)
            pltpu.sync_copy(acc.at[pl.ds(r1, ZR)], rows_b.at[pl.ds(0, ZR)])
            pltpu.async_copy(rows_b.at[pl.ds(0, ZR)],
                             po.at[g, cid, pl.ds(r1, ZR)], sem_b)
            pltpu.make_async_copy(rows_a.at[pl.ds(0, ZR)],
                                  po.at[g, cid, pl.ds(r0, ZR)],
                                  sem_a).wait()
            pltpu.make_async_copy(rows_b.at[pl.ds(0, ZR)],
                                  po.at[g, cid, pl.ds(r1, ZR)],
                                  sem_b).wait()
            return c
        lax.fori_loop(0, RT // ZR // 2, wout, 0)


_sc_call = pl.kernel(
    _sc_body,
    out_type=jax.ShapeDtypeStruct((3, NC, NP, D), jnp.bfloat16),
    mesh=plsc.VectorSubcoreMesh(core_axis_name="c", subcore_axis_name="s"),
    compiler_params=pltpu.CompilerParams(use_tc_tiling_on_sc=False),
    scratch_types=[
        pltpu.VMEM_SHARED((NP, D), jnp.bfloat16),
        pltpu.VMEM((CH2, K), jnp.int32),
        pltpu.VMEM((CH2, K), jnp.int32),
        pltpu.VMEM((K, D), jnp.bfloat16),
        pltpu.VMEM((K, D), jnp.bfloat16),
        pltpu.VMEM((K, D), jnp.bfloat16),
        pltpu.SemaphoreType.DMA,
        pltpu.SemaphoreType.DMA,
        pltpu.SemaphoreType.DMA,
    ],
)


def kernel(h_mf_new, h_bp_new, h_cc_new, mf_edge_index, bp_edge_index,
           cc_edge_index, W_mf, b_mf, W_bp, b_bp, W_cc, b_cc):
    bs = jnp.stack([b_mf, b_bp, b_cc]).reshape(3, 1, D)

    hw = _matmul(h_mf_new, h_bp_new, h_cc_new, W_mf, W_bp, W_cc)

    def _idx(ei):
        return ei.astype(jnp.int32).reshape(2, NC * NS, CH, K)

    parts = _sc_call(hw[0], hw[1], hw[2], _idx(mf_edge_index),
                     _idx(bp_edge_index), _idx(cc_edge_index))

    return tuple(_finalize(parts, h_mf_new, h_bp_new, h_cc_new, bs))
